# baseline (speedup 1.0000x reference)
import functools

import numpy as np
import jax
import jax.numpy as jnp
from jax import lax
from jax.experimental import pallas as pl
from jax.experimental.pallas import tpu as pltpu
from jax.experimental.pallas import tpu_sc as plsc

D_MODEL = 1024
MAX_LEN = 8192
BATCH = 4
SEQ_LEN = 4096
NUM_CORES = 2
NUM_SUBCORES = 16
NW = NUM_CORES * NUM_SUBCORES
S_PER_W = SEQ_LEN // NW
CHUNK = 32
NJ = S_PER_W // CHUNK
NITER = NJ * BATCH
LANES = 16


def _sinusoid_pe_np(max_len, d_model):
    pos = np.arange(max_len, dtype=np.float32)[:, None]
    i = np.arange(0, d_model, 2, dtype=np.float32)
    div = np.power(10000.0, i / d_model)
    pe = np.zeros((max_len, d_model), dtype=np.float32)
    pe[:, 0::2] = np.sin(pos / div)
    pe[:, 1::2] = np.cos(pos / div)
    return pe


_PE = _sinusoid_pe_np(MAX_LEN, D_MODEL)[:SEQ_LEN].astype(np.float32)


@functools.partial(
    pl.kernel,
    out_type=jax.ShapeDtypeStruct((BATCH * SEQ_LEN, D_MODEL), jnp.float32),
    mesh=plsc.VectorSubcoreMesh(core_axis_name="c", subcore_axis_name="s"),
    scratch_types=[
        pltpu.VMEM((BATCH, NJ, CHUNK), jnp.int32),
        pltpu.VMEM((CHUNK, D_MODEL), jnp.float32),
        pltpu.VMEM((CHUNK, D_MODEL), jnp.float32),
        pltpu.VMEM((CHUNK, D_MODEL), jnp.float32),
        pltpu.SemaphoreType.DMA,
        pltpu.SemaphoreType.DMA,
    ],
)
def _emb_kernel(x_hbm, table_hbm, pe_hbm, out_hbm,
                idx_v, rows0, rows1, pe_v, g0, g1):
    bufs = (rows0, rows1)
    gsems = (g0, g1)

    wid = lax.axis_index("s") * NUM_CORES + lax.axis_index("c")
    s_lo = wid * S_PER_W

    pltpu.sync_copy(x_hbm.at[wid], idx_v)

    def gather(n):
        b, j = n % BATCH, n // BATCH
        return pltpu.async_copy(
            table_hbm.at[idx_v.at[b, j]], bufs[n % 2], gsems[n % 2])

    ghandles = [gather(0), gather(1)]

    for n in range(NITER):
        b, j = n % BATCH, n // BATCH
        buf = bufs[n % 2]
        if b == 0:
            pltpu.sync_copy(pe_hbm.at[pl.ds(s_lo + j * CHUNK, CHUNK)], pe_v)
        ghandles[n % 2].wait()

        def add_row(r, carry):
            for c in range(D_MODEL // LANES):
                sl = pl.ds(c * LANES, LANES)
                plsc.addupdate(buf.at[r, sl], pe_v[r, sl])
            return carry

        lax.fori_loop(0, CHUNK, add_row, 0)
        pltpu.sync_copy(
            buf, out_hbm.at[pl.ds(b * SEQ_LEN + s_lo + j * CHUNK, CHUNK)])
        if n + 2 < NITER:
            ghandles[n % 2] = gather(n + 2)


def kernel(x, tok_table):
    x_grouped = x.reshape(BATCH, NW, NJ, CHUNK).transpose(1, 0, 2, 3)
    pe = jnp.asarray(_PE)
    out = _emb_kernel(x_grouped, tok_table, pe)
    return out.reshape(BATCH, SEQ_LEN, D_MODEL)


# speedup vs baseline: 1.3933x; 1.3933x over previous
import functools

import numpy as np
import jax
import jax.numpy as jnp
from jax import lax
from jax.experimental import pallas as pl
from jax.experimental.pallas import tpu as pltpu
from jax.experimental.pallas import tpu_sc as plsc

D_MODEL = 1024
MAX_LEN = 8192
BATCH = 4
SEQ_LEN = 4096
NUM_CORES = 2
NUM_SUBCORES = 16
NW = NUM_CORES * NUM_SUBCORES
S_PER_W = SEQ_LEN // NW
CHUNK = 32
NJ = S_PER_W // CHUNK
NITER = NJ * BATCH
LANES = 16


def _sinusoid_pe_np(max_len, d_model):
    pos = np.arange(max_len, dtype=np.float32)[:, None]
    i = np.arange(0, d_model, 2, dtype=np.float32)
    div = np.power(10000.0, i / d_model)
    pe = np.zeros((max_len, d_model), dtype=np.float32)
    pe[:, 0::2] = np.sin(pos / div)
    pe[:, 1::2] = np.cos(pos / div)
    return pe


_PE = _sinusoid_pe_np(MAX_LEN, D_MODEL)[:SEQ_LEN].astype(np.float32)


@functools.partial(
    pl.kernel,
    out_type=jax.ShapeDtypeStruct((BATCH * SEQ_LEN, D_MODEL), jnp.float32),
    mesh=plsc.VectorSubcoreMesh(core_axis_name="c", subcore_axis_name="s"),
    scratch_types=[
        pltpu.VMEM((BATCH, NJ, CHUNK), jnp.int32),
        pltpu.VMEM((CHUNK, D_MODEL), jnp.float32),
        pltpu.VMEM((CHUNK, D_MODEL), jnp.float32),
        pltpu.VMEM((CHUNK, D_MODEL), jnp.float32),
        pltpu.SemaphoreType.DMA,
        pltpu.SemaphoreType.DMA,
    ],
)
def _emb_kernel(x_hbm, table_hbm, pe_hbm, out_hbm,
                idx_v, rows0, rows1, pe_v, g0, g1):
    bufs = (rows0, rows1)
    gsems = (g0, g1)

    wid = lax.axis_index("s") * NUM_CORES + lax.axis_index("c")
    s_lo = wid * S_PER_W

    pltpu.sync_copy(x_hbm.at[wid], idx_v)

    def gather(n):
        b, j = n % BATCH, n // BATCH
        return pltpu.async_copy(
            table_hbm.at[idx_v.at[b, j]], bufs[n % 2], gsems[n % 2])

    ghandles = [gather(0), gather(1)]

    for n in range(NITER):
        b, j = n % BATCH, n // BATCH
        buf = bufs[n % 2]
        if b == 0:
            pltpu.sync_copy(pe_hbm.at[pl.ds(s_lo + j * CHUNK, CHUNK)], pe_v)
        ghandles[n % 2].wait()

        def add_row(r, carry):
            for c in range(D_MODEL // LANES):
                sl = pl.ds(c * LANES, LANES)
                buf[r, sl] = buf[r, sl] + pe_v[r, sl]
            return carry

        lax.fori_loop(0, CHUNK, add_row, 0)
        pltpu.sync_copy(
            buf, out_hbm.at[pl.ds(b * SEQ_LEN + s_lo + j * CHUNK, CHUNK)])
        if n + 2 < NITER:
            ghandles[n % 2] = gather(n + 2)


def kernel(x, tok_table):
    x_grouped = x.reshape(BATCH, NW, NJ, CHUNK).transpose(1, 0, 2, 3)
    pe = jnp.asarray(_PE)
    out = _emb_kernel(x_grouped, tok_table, pe)
    return out.reshape(BATCH, SEQ_LEN, D_MODEL)


# speedup vs baseline: 1.5433x; 1.1076x over previous
import functools

import numpy as np
import jax
import jax.numpy as jnp
from jax import lax
from jax.experimental import pallas as pl
from jax.experimental.pallas import tpu as pltpu
from jax.experimental.pallas import tpu_sc as plsc

D_MODEL = 1024
MAX_LEN = 8192
BATCH = 4
SEQ_LEN = 4096
NUM_CORES = 2
NUM_SUBCORES = 16
NW = NUM_CORES * NUM_SUBCORES
S_PER_W = SEQ_LEN // NW
CHUNK = 32
HALF = CHUNK // 2
NJ = S_PER_W // CHUNK
NITER = NJ * BATCH
LANES = 16


def _sinusoid_pe_np(max_len, d_model):
    pos = np.arange(max_len, dtype=np.float32)[:, None]
    i = np.arange(0, d_model, 2, dtype=np.float32)
    div = np.power(10000.0, i / d_model)
    pe = np.zeros((max_len, d_model), dtype=np.float32)
    pe[:, 0::2] = np.sin(pos / div)
    pe[:, 1::2] = np.cos(pos / div)
    return pe


_PE = _sinusoid_pe_np(MAX_LEN, D_MODEL)[:SEQ_LEN].astype(np.float32)


@functools.partial(
    pl.kernel,
    out_type=jax.ShapeDtypeStruct((BATCH * SEQ_LEN, D_MODEL), jnp.float32),
    mesh=plsc.VectorSubcoreMesh(core_axis_name="c", subcore_axis_name="s"),
    scratch_types=[
        pltpu.VMEM((BATCH, S_PER_W), jnp.int32),
        pltpu.VMEM((CHUNK, D_MODEL), jnp.float32),
        pltpu.VMEM((CHUNK, D_MODEL), jnp.float32),
        pltpu.VMEM((CHUNK, D_MODEL), jnp.float32),
        pltpu.SemaphoreType.DMA,
        pltpu.SemaphoreType.DMA,
        pltpu.SemaphoreType.DMA,
        pltpu.SemaphoreType.DMA,
        pltpu.SemaphoreType.DMA,
    ],
)
def _emb_kernel(x_hbm, table_hbm, pe_hbm, out_hbm,
                idx_v, rows0, rows1, pe_v, g0, g1, s0, s1, psem):
    bufs = (rows0, rows1)
    gsems = (g0, g1)
    ssems = (s0, s1)

    wid = lax.axis_index("s") * NUM_CORES + lax.axis_index("c")
    s_lo = wid * S_PER_W

    for b in range(BATCH):
        pltpu.sync_copy(x_hbm.at[b, pl.ds(s_lo, S_PER_W)], idx_v.at[b])

    def gather(n):
        b, j = n % BATCH, n // BATCH
        return pltpu.async_copy(
            table_hbm.at[idx_v.at[b, pl.ds(j * CHUNK, CHUNK)]],
            bufs[n % 2], gsems[n % 2])

    def pe_load(j):
        return pltpu.async_copy(
            pe_hbm.at[pl.ds(s_lo + j * CHUNK, CHUNK)], pe_v, psem)

    def add_rows(buf, lo, hi):
        def add_row(r, carry):
            for c in range(D_MODEL // LANES):
                sl = pl.ds(c * LANES, LANES)
                buf[r, sl] = buf[r, sl] + pe_v[r, sl]
            return carry

        lax.fori_loop(lo, hi, add_row, 0)

    def store_half(n, h):
        b, j = n % BATCH, n // BATCH
        row0 = b * SEQ_LEN + s_lo + j * CHUNK + h * HALF
        return pltpu.async_copy(
            bufs[n % 2].at[pl.ds(h * HALF, HALF)],
            out_hbm.at[pl.ds(row0, HALF)], ssems[n % 2])

    ghandle = gather(0)
    phandle = pe_load(0)
    prev_stores = None

    for n in range(NITER):
        b, j = n % BATCH, n // BATCH
        buf = bufs[n % 2]
        if b == 0:
            phandle.wait()
        ghandle.wait()
        add_rows(buf, 0, HALF)
        sh0 = store_half(n, 0)
        if prev_stores is not None:
            prev_stores[0].wait()
            prev_stores[1].wait()
        if n + 1 < NITER:
            ghandle = gather(n + 1)
        add_rows(buf, HALF, CHUNK)
        sh1 = store_half(n, 1)
        if b == BATCH - 1 and j + 1 < NJ:
            phandle = pe_load(j + 1)
        prev_stores = (sh0, sh1)

    prev_stores[0].wait()
    prev_stores[1].wait()


def kernel(x, tok_table):
    pe = jnp.asarray(_PE)
    out = _emb_kernel(x, tok_table, pe)
    return out.reshape(BATCH, SEQ_LEN, D_MODEL)


# speedup vs baseline: 1.5534x; 1.0066x over previous
import functools

import numpy as np
import jax
import jax.numpy as jnp
from jax import lax
from jax.experimental import pallas as pl
from jax.experimental.pallas import tpu as pltpu
from jax.experimental.pallas import tpu_sc as plsc

D_MODEL = 1024
MAX_LEN = 8192
BATCH = 4
SEQ_LEN = 4096
NUM_CORES = 2
NUM_SUBCORES = 16
NW = NUM_CORES * NUM_SUBCORES
S_PER_W = SEQ_LEN // NW
CHUNK = 32
HALF = CHUNK // 2
NJ = S_PER_W // CHUNK
NITER = NJ * BATCH
LANES = 16


def _sinusoid_pe_np(max_len, d_model):
    pos = np.arange(max_len, dtype=np.float32)[:, None]
    i = np.arange(0, d_model, 2, dtype=np.float32)
    div = np.power(10000.0, i / d_model)
    pe = np.zeros((max_len, d_model), dtype=np.float32)
    pe[:, 0::2] = np.sin(pos / div)
    pe[:, 1::2] = np.cos(pos / div)
    return pe


_PE = _sinusoid_pe_np(MAX_LEN, D_MODEL)[:SEQ_LEN].astype(np.float32)


@functools.partial(
    pl.kernel,
    out_type=jax.ShapeDtypeStruct((BATCH * SEQ_LEN, D_MODEL), jnp.float32),
    mesh=plsc.VectorSubcoreMesh(core_axis_name="c", subcore_axis_name="s"),
    scratch_types=[
        pltpu.VMEM((BATCH, S_PER_W), jnp.int32),
        pltpu.VMEM((CHUNK, D_MODEL), jnp.float32),
        pltpu.VMEM((CHUNK, D_MODEL), jnp.float32),
        pltpu.VMEM((CHUNK, D_MODEL), jnp.float32),
        pltpu.SemaphoreType.DMA,
        pltpu.SemaphoreType.DMA,
        pltpu.SemaphoreType.DMA,
        pltpu.SemaphoreType.DMA,
        pltpu.SemaphoreType.DMA,
    ],
)
def _emb_kernel(x_hbm, table_hbm, pe_hbm, out_hbm,
                idx_v, rows0, rows1, pe_v, g0, g1, s0, s1, psem):
    bufs = (rows0, rows1)
    gsems = (g0, g1)
    ssems = (s0, s1)

    wid = lax.axis_index("s") * NUM_CORES + lax.axis_index("c")
    s_lo = wid * S_PER_W

    pltpu.sync_copy(x_hbm.at[pl.ds(0, BATCH), pl.ds(s_lo, S_PER_W)], idx_v)

    def gather(n):
        b, j = n % BATCH, n // BATCH
        return pltpu.async_copy(
            table_hbm.at[idx_v.at[b, pl.ds(j * CHUNK, CHUNK)]],
            bufs[n % 2], gsems[n % 2])

    def pe_load(j):
        return pltpu.async_copy(
            pe_hbm.at[pl.ds(s_lo + j * CHUNK, CHUNK)], pe_v, psem)

    def add_rows(buf, lo, hi):
        def add_row(r, carry):
            for c in range(D_MODEL // LANES):
                sl = pl.ds(c * LANES, LANES)
                buf[r, sl] = buf[r, sl] + pe_v[r, sl]
            return carry

        lax.fori_loop(lo, hi, add_row, 0)

    def store_half(n, h):
        b, j = n % BATCH, n // BATCH
        row0 = b * SEQ_LEN + s_lo + j * CHUNK + h * HALF
        return pltpu.async_copy(
            bufs[n % 2].at[pl.ds(h * HALF, HALF)],
            out_hbm.at[pl.ds(row0, HALF)], ssems[n % 2])

    ghandle = gather(0)
    phandle = pe_load(0)
    prev_stores = None

    for n in range(NITER):
        b, j = n % BATCH, n // BATCH
        buf = bufs[n % 2]
        if b == 0:
            phandle.wait()
        ghandle.wait()
        add_rows(buf, 0, HALF)
        sh0 = store_half(n, 0)
        if prev_stores is not None:
            prev_stores[0].wait()
            prev_stores[1].wait()
        if n + 1 < NITER:
            ghandle = gather(n + 1)
        add_rows(buf, HALF, CHUNK)
        sh1 = store_half(n, 1)
        if b == BATCH - 1 and j + 1 < NJ:
            phandle = pe_load(j + 1)
        prev_stores = (sh0, sh1)

    prev_stores[0].wait()
    prev_stores[1].wait()


def kernel(x, tok_table):
    pe = jnp.asarray(_PE)
    out = _emb_kernel(x, tok_table, pe)
    return out.reshape(BATCH, SEQ_LEN, D_MODEL)


# speedup vs baseline: 1.5675x; 1.0090x over previous
import functools

import numpy as np
import jax
import jax.numpy as jnp
from jax import lax
from jax.experimental import pallas as pl
from jax.experimental.pallas import tpu as pltpu
from jax.experimental.pallas import tpu_sc as plsc

D_MODEL = 1024
MAX_LEN = 8192
BATCH = 4
SEQ_LEN = 4096
NUM_CORES = 2
NUM_SUBCORES = 16
NW = NUM_CORES * NUM_SUBCORES
S_PER_W = SEQ_LEN // NW
CHUNK = 32
HALF = CHUNK // 2
NJ = S_PER_W // CHUNK
NITER = NJ * BATCH
LANES = 16


def _sinusoid_pe_np(max_len, d_model):
    pos = np.arange(max_len, dtype=np.float32)[:, None]
    i = np.arange(0, d_model, 2, dtype=np.float32)
    div = np.power(10000.0, i / d_model)
    pe = np.zeros((max_len, d_model), dtype=np.float32)
    pe[:, 0::2] = np.sin(pos / div)
    pe[:, 1::2] = np.cos(pos / div)
    return pe


_PE = _sinusoid_pe_np(MAX_LEN, D_MODEL)[:SEQ_LEN].astype(np.float32)


@functools.partial(
    pl.kernel,
    out_type=jax.ShapeDtypeStruct((BATCH * SEQ_LEN, D_MODEL), jnp.float32),
    mesh=plsc.VectorSubcoreMesh(core_axis_name="c", subcore_axis_name="s"),
    scratch_types=[
        pltpu.VMEM((BATCH, S_PER_W), jnp.int32),
        pltpu.VMEM((CHUNK, D_MODEL), jnp.float32),
        pltpu.VMEM((CHUNK, D_MODEL), jnp.float32),
        pltpu.VMEM((CHUNK, D_MODEL), jnp.float32),
        pltpu.SemaphoreType.DMA,
        pltpu.SemaphoreType.DMA,
        pltpu.SemaphoreType.DMA,
        pltpu.SemaphoreType.DMA,
        pltpu.SemaphoreType.DMA,
    ],
)
def _emb_kernel(x_hbm, table_hbm, pe_hbm, out_hbm,
                idx_v, rows0, rows1, pe_v, g0, g1, s0, s1, psem):
    bufs = (rows0, rows1)
    gsems = (g0, g1)
    ssems = (s0, s1)

    wid = lax.axis_index("s") * NUM_CORES + lax.axis_index("c")
    s_lo = wid * S_PER_W

    pltpu.sync_copy(x_hbm.at[pl.ds(0, BATCH), pl.ds(s_lo, S_PER_W)], idx_v)

    def gather(n):
        b, j = n % BATCH, n // BATCH
        return pltpu.async_copy(
            table_hbm.at[idx_v.at[b, pl.ds(j * CHUNK, CHUNK)]],
            bufs[n % 2], gsems[n % 2])

    def pe_load(j):
        return pltpu.async_copy(
            pe_hbm.at[pl.ds(s_lo + j * CHUNK, CHUNK)], pe_v, psem)

    def add_rows(buf, lo, hi):
        def add_row(r, carry):
            for c in range(D_MODEL // LANES):
                sl = pl.ds(c * LANES, LANES)
                buf[r, sl] = buf[r, sl] + pe_v[r, sl]
            return carry

        lax.fori_loop(lo, hi, add_row, 0)

    def store_half(n, h):
        b, j = n % BATCH, n // BATCH
        row0 = b * SEQ_LEN + s_lo + j * CHUNK + h * HALF
        return pltpu.async_copy(
            bufs[n % 2].at[pl.ds(h * HALF, HALF)],
            out_hbm.at[pl.ds(row0, HALF)], ssems[n % 2])

    ghandle = gather(0)
    phandle = pe_load(0)
    prev_stores = None

    for n in range(NITER):
        b, j = n % BATCH, n // BATCH
        buf = bufs[n % 2]
        if b == 0:
            phandle.wait()
        if prev_stores is not None:
            prev_stores[0].wait()
            prev_stores[1].wait()
        if n + 1 < NITER:
            ghandle_next = gather(n + 1)
        ghandle.wait()
        add_rows(buf, 0, HALF)
        sh0 = store_half(n, 0)
        add_rows(buf, HALF, CHUNK)
        sh1 = store_half(n, 1)
        if n + 1 < NITER:
            ghandle = ghandle_next
        if b == BATCH - 1 and j + 1 < NJ:
            phandle = pe_load(j + 1)
        prev_stores = (sh0, sh1)

    prev_stores[0].wait()
    prev_stores[1].wait()


def kernel(x, tok_table):
    pe = jnp.asarray(_PE)
    out = _emb_kernel(x, tok_table, pe)
    return out.reshape(BATCH, SEQ_LEN, D_MODEL)


# speedup vs baseline: 1.5689x; 1.0009x over previous
import functools

import numpy as np
import jax
import jax.numpy as jnp
from jax import lax
from jax.experimental import pallas as pl
from jax.experimental.pallas import tpu as pltpu
from jax.experimental.pallas import tpu_sc as plsc

D_MODEL = 1024
MAX_LEN = 8192
BATCH = 4
SEQ_LEN = 4096
NUM_CORES = 2
NUM_SUBCORES = 16
NW = NUM_CORES * NUM_SUBCORES
S_PER_W = SEQ_LEN // NW
CHUNK = 32
HALF = CHUNK // 2
NJ = S_PER_W // CHUNK
NITER = NJ * BATCH
LANES = 16


def _sinusoid_pe_np(max_len, d_model):
    pos = np.arange(max_len, dtype=np.float32)[:, None]
    i = np.arange(0, d_model, 2, dtype=np.float32)
    div = np.power(10000.0, i / d_model)
    pe = np.zeros((max_len, d_model), dtype=np.float32)
    pe[:, 0::2] = np.sin(pos / div)
    pe[:, 1::2] = np.cos(pos / div)
    return pe


_PE = _sinusoid_pe_np(MAX_LEN, D_MODEL)[:SEQ_LEN].astype(np.float32)


@functools.partial(
    pl.kernel,
    out_type=jax.ShapeDtypeStruct((BATCH * SEQ_LEN, D_MODEL), jnp.float32),
    mesh=plsc.VectorSubcoreMesh(core_axis_name="c", subcore_axis_name="s"),
    scratch_types=[
        pltpu.VMEM((BATCH, S_PER_W), jnp.int32),
        pltpu.VMEM((CHUNK, D_MODEL), jnp.float32),
        pltpu.VMEM((CHUNK, D_MODEL), jnp.float32),
        pltpu.VMEM((CHUNK, D_MODEL), jnp.float32),
        pltpu.SemaphoreType.DMA,
        pltpu.SemaphoreType.DMA,
        pltpu.SemaphoreType.DMA,
        pltpu.SemaphoreType.DMA,
        pltpu.SemaphoreType.DMA,
        pltpu.SemaphoreType.DMA,
        pltpu.SemaphoreType.DMA,
    ],
)
def _emb_kernel(x_hbm, table_hbm, pe_hbm, out_hbm,
                idx_v, rows0, rows1, pe_v,
                g00, g01, g10, g11, s0, s1, psem):
    bufs = (rows0, rows1)
    gsems = ((g00, g01), (g10, g11))
    ssems = (s0, s1)

    wid = lax.axis_index("s") * NUM_CORES + lax.axis_index("c")
    s_lo = wid * S_PER_W

    pltpu.sync_copy(x_hbm.at[pl.ds(0, BATCH), pl.ds(s_lo, S_PER_W)], idx_v)

    def gather_half(n, h):
        b, j = n % BATCH, n // BATCH
        return pltpu.async_copy(
            table_hbm.at[idx_v.at[b, pl.ds(j * CHUNK + h * HALF, HALF)]],
            bufs[n % 2].at[pl.ds(h * HALF, HALF)], gsems[n % 2][h])

    def pe_load(j):
        return pltpu.async_copy(
            pe_hbm.at[pl.ds(s_lo + j * CHUNK, CHUNK)], pe_v, psem)

    def add_rows(buf, lo, hi):
        def add_row(r, carry):
            for c in range(D_MODEL // LANES):
                sl = pl.ds(c * LANES, LANES)
                buf[r, sl] = buf[r, sl] + pe_v[r, sl]
            return carry

        lax.fori_loop(lo, hi, add_row, 0)

    def store_half(n, h):
        b, j = n % BATCH, n // BATCH
        row0 = b * SEQ_LEN + s_lo + j * CHUNK + h * HALF
        return pltpu.async_copy(
            bufs[n % 2].at[pl.ds(h * HALF, HALF)],
            out_hbm.at[pl.ds(row0, HALF)], ssems[n % 2])

    ghandles = (gather_half(0, 0), gather_half(0, 1))
    phandle = pe_load(0)
    prev_stores = None

    for n in range(NITER):
        b, j = n % BATCH, n // BATCH
        buf = bufs[n % 2]
        if b == 0:
            phandle.wait()
        gh0 = gh1 = None
        if prev_stores is not None:
            prev_stores[0].wait()
            if n + 1 < NITER:
                gh0 = gather_half(n + 1, 0)
            prev_stores[1].wait()
            if n + 1 < NITER:
                gh1 = gather_half(n + 1, 1)
        else:
            gh0 = gather_half(n + 1, 0)
            gh1 = gather_half(n + 1, 1)
        ghandles[0].wait()
        add_rows(buf, 0, HALF)
        sh0 = store_half(n, 0)
        ghandles[1].wait()
        add_rows(buf, HALF, CHUNK)
        sh1 = store_half(n, 1)
        if b == BATCH - 1 and j + 1 < NJ:
            phandle = pe_load(j + 1)
        prev_stores = (sh0, sh1)
        ghandles = (gh0, gh1)

    prev_stores[0].wait()
    prev_stores[1].wait()


def kernel(x, tok_table):
    pe = jnp.asarray(_PE)
    out = _emb_kernel(x, tok_table, pe)
    return out.reshape(BATCH, SEQ_LEN, D_MODEL)


# speedup vs baseline: 1.6790x; 1.0702x over previous
import functools

import numpy as np
import jax
import jax.numpy as jnp
from jax import lax
from jax.experimental import pallas as pl
from jax.experimental.pallas import tpu as pltpu
from jax.experimental.pallas import tpu_sc as plsc

D_MODEL = 1024
MAX_LEN = 8192
BATCH = 4
SEQ_LEN = 4096
NUM_CORES = 2
NUM_SUBCORES = 16
NW = NUM_CORES * NUM_SUBCORES
S_PER_W = SEQ_LEN // NW
CHUNK = 32
HALF = CHUNK // 2
NJ = S_PER_W // CHUNK
NITER = NJ * BATCH
LANES = 16


def _sinusoid_pe_np(max_len, d_model):
    pos = np.arange(max_len, dtype=np.float32)[:, None]
    i = np.arange(0, d_model, 2, dtype=np.float32)
    div = np.power(10000.0, i / d_model)
    pe = np.zeros((max_len, d_model), dtype=np.float32)
    pe[:, 0::2] = np.sin(pos / div)
    pe[:, 1::2] = np.cos(pos / div)
    return pe


_PE = _sinusoid_pe_np(MAX_LEN, D_MODEL)[:SEQ_LEN].astype(np.float32)


@functools.partial(
    pl.kernel,
    out_type=jax.ShapeDtypeStruct((BATCH * SEQ_LEN, D_MODEL), jnp.float32),
    mesh=plsc.VectorSubcoreMesh(core_axis_name="c", subcore_axis_name="s"),
    scratch_types=[
        pltpu.VMEM((BATCH, S_PER_W), jnp.int32),
        pltpu.VMEM((CHUNK, D_MODEL), jnp.float32),
        pltpu.VMEM((CHUNK, D_MODEL), jnp.float32),
        pltpu.VMEM((CHUNK, D_MODEL), jnp.float32),
        pltpu.SemaphoreType.DMA,
        pltpu.SemaphoreType.DMA,
        pltpu.SemaphoreType.DMA,
        pltpu.SemaphoreType.DMA,
        pltpu.SemaphoreType.DMA,
        pltpu.SemaphoreType.DMA,
        pltpu.SemaphoreType.DMA,
    ],
)
def _emb_kernel(x_hbm, table_hbm, pe_hbm, out_hbm,
                idx_v, rows0, rows1, pe_v,
                g00, g01, g10, g11, s0, s1, psem):
    bufs = (rows0, rows1)
    gsems = ((g00, g01), (g10, g11))
    ssems = (s0, s1)

    wid = lax.axis_index("s") * NUM_CORES + lax.axis_index("c")
    s_lo = wid * S_PER_W

    pltpu.sync_copy(x_hbm.at[pl.ds(0, BATCH), pl.ds(s_lo, S_PER_W)], idx_v)

    def gather_half(n, h):
        b, j = n % BATCH, n // BATCH
        return pltpu.async_copy(
            table_hbm.at[idx_v.at[b, pl.ds(j * CHUNK + h * HALF, HALF)]],
            bufs[n % 2].at[pl.ds(h * HALF, HALF)], gsems[n % 2][h])

    def pe_load(j):
        return pltpu.async_copy(
            pe_hbm.at[pl.ds(s_lo + j * CHUNK, CHUNK)], pe_v, psem)

    def add_rows(buf, lo, hi):
        def add_row(r, carry):
            for c in range(D_MODEL // LANES):
                sl = pl.ds(c * LANES, LANES)
                buf[r, sl] = buf[r, sl] + pe_v[r, sl]
            return carry

        lax.fori_loop(lo, hi, add_row, 0)

    def store_half(n, h):
        b, j = n % BATCH, n // BATCH
        row0 = b * SEQ_LEN + s_lo + j * CHUNK + h * HALF
        return pltpu.async_copy(
            bufs[n % 2].at[pl.ds(h * HALF, HALF)],
            out_hbm.at[pl.ds(row0, HALF)], ssems[n % 2])

    ghandles = (gather_half(0, 0), gather_half(0, 1))
    phandle = pe_load(0)
    prev_stores = None

    for n in range(NITER):
        b, j = n % BATCH, n // BATCH
        buf = bufs[n % 2]
        if b == 0:
            phandle.wait()
        gh0 = gh1 = None
        if prev_stores is not None:
            prev_stores[0].wait()
            if n + 1 < NITER:
                gh0 = gather_half(n + 1, 0)
        else:
            gh0 = gather_half(n + 1, 0)
            gh1 = gather_half(n + 1, 1)
        ghandles[0].wait()
        add_rows(buf, 0, HALF)
        sh0 = store_half(n, 0)
        if prev_stores is not None:
            prev_stores[1].wait()
            if n + 1 < NITER:
                gh1 = gather_half(n + 1, 1)
        ghandles[1].wait()
        add_rows(buf, HALF, CHUNK)
        sh1 = store_half(n, 1)
        if b == BATCH - 1 and j + 1 < NJ:
            phandle = pe_load(j + 1)
        prev_stores = (sh0, sh1)
        ghandles = (gh0, gh1)

    prev_stores[0].wait()
    prev_stores[1].wait()


def kernel(x, tok_table):
    pe = jnp.asarray(_PE)
    out = _emb_kernel(x, tok_table, pe)
    return out.reshape(BATCH, SEQ_LEN, D_MODEL)
